# baseline (speedup 1.0000x reference)
import numpy as np

import jax
import jax.numpy as jnp
from jax.experimental import pallas as pl
from jax.experimental.pallas import tpu as pltpu


def _conv1_selector():
    h = np.arange(8)
    w = np.arange(28)
    p = np.arange(2)
    q = np.arange(12)
    k = np.arange(5)
    mats = []
    for a in (0, 1):
        A = (h[:, None, None] == 2 * p[None, :, None] + a + k[None, None, :])
        for b in (0, 1):
            Bm = (w[:, None, None] == 2 * q[None, :, None] + b + k[None, None, :])
            m = np.einsum('hpk,wqm->hwpqkm', A, Bm).reshape(5376, 25)
            mats.append(m)
    return np.stack(mats).astype(np.float32)


def _conv2_selector():
    ph = np.arange(6)
    q = np.arange(12)
    x = np.arange(4)
    k = np.arange(5)
    mats = []
    for a in (0, 1):
        C = (k[None, :] == ph[:, None] - a)
        for b in (0, 1):
            D = (q[:, None, None] == 2 * x[None, :, None] + b + k[None, None, :])
            m = np.einsum('pk,qxm->pqxkm', C, D).reshape(288, 25)
            mats.append(m)
    return np.stack(mats).astype(np.float32)


_SEL1 = _conv1_selector()
_SEL2 = _conv2_selector()


def _net_kernel(x_ref, w1_ref, w2_ref, wf_ref, bias_ref, o_ref):
    tb = x_ref.shape[0]
    xh = x_ref[...].astype(jnp.bfloat16)
    b1row = bias_ref[:, :512]
    b2row = bias_ref[:, 512:768]
    f1b = bias_ref[:, 768:1024]
    f2b = bias_ref[:, 1024:1034]

    x1 = jnp.concatenate([xh[:, 112 * g:112 * g + 224] for g in range(6)],
                         axis=0)
    z1 = jnp.dot(x1, w1_ref[...], preferred_element_type=jnp.float32)
    a1_bands = []
    for g in range(6):
        z = z1[tb * g:tb * (g + 1)]
        m = jnp.maximum(jnp.maximum(z[:, :512], z[:, 512:1024]),
                        jnp.maximum(z[:, 1024:1536], z[:, 1536:]))
        a1_bands.append(
            jnp.maximum(m + b1row, 0.0).astype(jnp.bfloat16))
    a1 = jnp.concatenate(a1_bands, axis=1)

    l2 = jnp.concatenate([a1[:, 512 * y:512 * y + 1536] for y in range(4)],
                         axis=0)
    z2 = jnp.dot(l2, w2_ref[...], preferred_element_type=jnp.float32)
    a2_bands = []
    for y in range(4):
        z = z2[tb * y:tb * (y + 1)]
        m = jnp.maximum(jnp.maximum(z[:, :256], z[:, 256:512]),
                        jnp.maximum(z[:, 512:768], z[:, 768:]))
        a2_bands.append(
            jnp.maximum(m + b2row, 0.0).astype(jnp.bfloat16))
    a2 = jnp.concatenate(a2_bands, axis=1)

    h = jnp.dot(a2, wf_ref[:1024], preferred_element_type=jnp.float32)
    h = jnp.maximum(h + f1b, 0.0).astype(jnp.bfloat16)
    y = jnp.dot(h, wf_ref[1024:1280, :10],
                preferred_element_type=jnp.float32)
    y = y + f2b
    m = jnp.max(y, axis=-1, keepdims=True)
    s = y - m
    lse = jnp.log(jnp.sum(jnp.exp(s), axis=-1, keepdims=True))
    o_ref[...] = (s - lse).astype(o_ref.dtype)


def _spread_conv1(w1):
    t = jnp.dot(jnp.asarray(_SEL1).reshape(4 * 5376, 25), w1)
    t = t.reshape(4, 8, 28, 2, 12, 20)
    t = jnp.transpose(t, (1, 2, 0, 3, 4, 5)).reshape(8, 28, 4, 480)
    t = jnp.pad(t, ((0, 0), (0, 0), (0, 0), (0, 32)))
    return t.reshape(224, 2048).astype(jnp.bfloat16)


def _spread_conv2(w2):
    w2r = w2.reshape(25, 1000)
    t = jnp.dot(jnp.asarray(_SEL2).reshape(4 * 288, 25), w2r)
    t = t.reshape(4, 6, 12, 4, 20, 50)
    t = jnp.transpose(t, (0, 1, 2, 4, 3, 5))
    t = t.reshape(4, 3, 480, 200)
    t = jnp.pad(t, ((0, 0), (0, 0), (0, 32), (0, 56)))
    t = jnp.transpose(t.reshape(4, 1536, 256), (1, 0, 2))
    return t.reshape(1536, 1024).astype(jnp.bfloat16)


def kernel(x, w1, b1, w2, b2, fc1_w, fc1_b, fc2_w, fc2_b):
    B = x.shape[0]
    xp = x.reshape(B, 784)

    W1 = _spread_conv1(w1)
    W2 = _spread_conv2(w2)
    f1w = jnp.pad(fc1_w.reshape(4, 200, 256),
                  ((0, 0), (0, 56), (0, 0))).reshape(1024, 256)
    wf = jnp.concatenate(
        [f1w, jnp.pad(fc2_w, ((0, 0), (0, 246)))],
        axis=0).astype(jnp.bfloat16)
    biases = jnp.concatenate(
        [jnp.pad(jnp.tile(b1, (1, 24)), ((0, 0), (0, 32))),
         jnp.pad(jnp.tile(b2, (1, 4)), ((0, 0), (0, 56))),
         fc1_b, fc2_b], axis=1)

    TB = next((t for t in (256, 128, 64, 32, 16, 8) if B % t == 0), B)
    row_spec = lambda shape: pl.BlockSpec(shape, lambda i: (i, 0))
    full_spec = lambda shape: pl.BlockSpec(shape, lambda i: (0, 0))

    return pl.pallas_call(
        _net_kernel,
        out_shape=jax.ShapeDtypeStruct((B, 10), jnp.float32),
        grid=(B // TB,),
        in_specs=[
            row_spec((TB, 784)),
            full_spec(W1.shape),
            full_spec(W2.shape),
            full_spec(wf.shape),
            full_spec(biases.shape),
        ],
        out_specs=row_spec((TB, 10)),
        compiler_params=pltpu.CompilerParams(
            dimension_semantics=("parallel",),
            vmem_limit_bytes=100 * 1024 * 1024,
        ),
    )(xp, W1, W2, wf, biases)


# speedup vs baseline: 1.0170x; 1.0170x over previous
import jax
import jax.numpy as jnp
from jax.experimental import pallas as pl
from jax.experimental.pallas import tpu as pltpu


def _wspread(src, rows, out_w, shift):
    taps = src.shape[0]
    width = out_w + 2
    trail = src.shape[1:]
    bc = jnp.broadcast_to(src[None], (rows,) + src.shape)
    bc = jnp.pad(bc.reshape(rows, taps, -1),
                 ((0, 0), (0, width - taps), (0, 0)))
    flat = bc.reshape(rows * width, -1)
    flat = jnp.pad(flat, ((shift, 0), (0, 0)))[:rows * out_w]
    return flat.reshape((rows, out_w) + trail)


def _net_kernel(x_ref, w1_ref, w2_ref, wf_ref, bias_ref, o_ref):
    tb = x_ref.shape[0]
    xh = x_ref[...].astype(jnp.bfloat16)
    b1row = bias_ref[:, :512]
    b2row = bias_ref[:, 512:768]
    f1b = bias_ref[:, 768:1024]
    f2b = bias_ref[:, 1024:1034]

    x1 = jnp.concatenate([xh[:, 112 * g:112 * g + 224] for g in range(6)],
                         axis=0)
    z1 = jnp.dot(x1, w1_ref[...], preferred_element_type=jnp.float32)
    a1_bands = []
    for g in range(6):
        z = z1[tb * g:tb * (g + 1)]
        m = jnp.maximum(jnp.maximum(z[:, :512], z[:, 512:1024]),
                        jnp.maximum(z[:, 1024:1536], z[:, 1536:]))
        a1_bands.append(
            jnp.maximum(m + b1row, 0.0).astype(jnp.bfloat16))
    a1 = jnp.concatenate(a1_bands, axis=1)

    l2 = jnp.concatenate([a1[:, 512 * y:512 * y + 1536] for y in range(4)],
                         axis=0)
    z2 = jnp.dot(l2, w2_ref[...], preferred_element_type=jnp.float32)
    a2_bands = []
    for y in range(4):
        z = z2[tb * y:tb * (y + 1)]
        m = jnp.maximum(jnp.maximum(z[:, :256], z[:, 256:512]),
                        jnp.maximum(z[:, 512:768], z[:, 768:]))
        a2_bands.append(
            jnp.maximum(m + b2row, 0.0).astype(jnp.bfloat16))
    a2 = jnp.concatenate(a2_bands, axis=1)

    h = jnp.dot(a2, wf_ref[:1024], preferred_element_type=jnp.float32)
    h = jnp.maximum(h + f1b, 0.0).astype(jnp.bfloat16)
    y = jnp.dot(h, wf_ref[1024:1280, :10],
                preferred_element_type=jnp.float32)
    y = y + f2b
    m = jnp.max(y, axis=-1, keepdims=True)
    s = y - m
    lse = jnp.log(jnp.sum(jnp.exp(s), axis=-1, keepdims=True))
    o_ref[...] = (s - lse).astype(o_ref.dtype)


def _spread_conv1(w1):
    w15 = w1.reshape(5, 5, 20)
    s1 = jnp.transpose(w15, (1, 0, 2))
    blocks = []
    for a in (0, 1):
        for b in (0, 1):
            v = _wspread(s1, 12, 28, b)
            v = jnp.transpose(v, (2, 0, 1, 3))
            t = _wspread(v, 2, 8, a)
            t = jnp.transpose(t, (1, 3, 0, 2, 4))
            blocks.append(jnp.pad(t.reshape(224, 480), ((0, 0), (0, 32))))
    return jnp.concatenate(blocks, axis=1).astype(jnp.bfloat16)


def _spread_conv2(w2):
    w25 = w2.reshape(5, 5, 20, 50)
    s1 = jnp.transpose(w25, (1, 0, 2, 3))
    blocks = []
    for a in (0, 1):
        for b in (0, 1):
            v = _wspread(s1, 4, 12, b)
            v = jnp.pad(v, ((0, 0), (0, 0), (a, 1 - a), (0, 0), (0, 0)))
            t = jnp.transpose(v, (2, 1, 3, 0, 4))
            t = t.reshape(3, 480, 200)
            t = jnp.pad(t, ((0, 0), (0, 32), (0, 56)))
            blocks.append(t.reshape(1536, 256))
    return jnp.concatenate(blocks, axis=1).astype(jnp.bfloat16)


def kernel(x, w1, b1, w2, b2, fc1_w, fc1_b, fc2_w, fc2_b):
    B = x.shape[0]
    xp = x.reshape(B, 784)

    W1 = _spread_conv1(w1)
    W2 = _spread_conv2(w2)
    f1w = jnp.pad(fc1_w.reshape(4, 200, 256),
                  ((0, 0), (0, 56), (0, 0))).reshape(1024, 256)
    wf = jnp.concatenate(
        [f1w, jnp.pad(fc2_w, ((0, 0), (0, 246)))],
        axis=0).astype(jnp.bfloat16)
    biases = jnp.concatenate(
        [jnp.pad(jnp.tile(b1, (1, 24)), ((0, 0), (0, 32))),
         jnp.pad(jnp.tile(b2, (1, 4)), ((0, 0), (0, 56))),
         fc1_b, fc2_b], axis=1)

    TB = next((t for t in (256, 128, 64, 32, 16, 8) if B % t == 0), B)
    row_spec = lambda shape: pl.BlockSpec(shape, lambda i: (i, 0))
    full_spec = lambda shape: pl.BlockSpec(shape, lambda i: (0, 0))

    return pl.pallas_call(
        _net_kernel,
        out_shape=jax.ShapeDtypeStruct((B, 10), jnp.float32),
        grid=(B // TB,),
        in_specs=[
            row_spec((TB, 784)),
            full_spec(W1.shape),
            full_spec(W2.shape),
            full_spec(wf.shape),
            full_spec(biases.shape),
        ],
        out_specs=row_spec((TB, 10)),
        compiler_params=pltpu.CompilerParams(
            dimension_semantics=("parallel",),
            vmem_limit_bytes=100 * 1024 * 1024,
        ),
    )(xp, W1, W2, wf, biases)


# speedup vs baseline: 1.0217x; 1.0047x over previous
import jax
import jax.numpy as jnp
from jax.experimental import pallas as pl
from jax.experimental.pallas import tpu as pltpu


def _wspread(src, rows, out_w, shift):
    taps = src.shape[0]
    width = out_w + 2
    trail = src.shape[1:]
    bc = jnp.broadcast_to(src[None], (rows,) + src.shape)
    bc = jnp.pad(bc.reshape(rows, taps, -1),
                 ((0, 0), (0, width - taps), (0, 0)))
    flat = bc.reshape(rows * width, -1)
    flat = jnp.pad(flat, ((shift, 0), (0, 0)))[:rows * out_w]
    return flat.reshape((rows, out_w) + trail)


def _net_kernel(x_ref, w1_ref, w2_ref, wf_ref, bias_ref, o_ref):
    tb = x_ref.shape[0]
    xh = x_ref[...].astype(jnp.bfloat16)
    b1row = bias_ref[:, :512]
    b2row = bias_ref[:, 512:768]
    f1b = bias_ref[:, 768:1024]
    f2b = bias_ref[:, 1024:1034]

    x1 = jnp.concatenate([xh[:, 112 * g:112 * g + 224] for g in range(6)],
                         axis=0)
    z1 = jnp.dot(x1, w1_ref[...], preferred_element_type=jnp.float32)
    a1_bands = []
    for g in range(6):
        z = z1[tb * g:tb * (g + 1)]
        m = jnp.maximum(jnp.maximum(z[:, :512], z[:, 512:1024]),
                        jnp.maximum(z[:, 1024:1536], z[:, 1536:]))
        a1_bands.append(
            jnp.maximum(m + b1row, 0.0).astype(jnp.bfloat16))
    a1 = jnp.concatenate(a1_bands, axis=1)

    l2 = jnp.concatenate([a1[:, 512 * y:512 * y + 1536] for y in range(4)],
                         axis=0)
    z2 = jnp.dot(l2, w2_ref[...], preferred_element_type=jnp.float32)
    a2_bands = []
    for y in range(4):
        z = z2[tb * y:tb * (y + 1)]
        m = jnp.maximum(jnp.maximum(z[:, :256], z[:, 256:512]),
                        jnp.maximum(z[:, 512:768], z[:, 768:]))
        a2_bands.append(
            jnp.maximum(m + b2row, 0.0).astype(jnp.bfloat16))
    a2 = jnp.concatenate(a2_bands, axis=1)

    h = jnp.dot(a2, wf_ref[:1024], preferred_element_type=jnp.float32)
    h = jnp.maximum(h + f1b, 0.0).astype(jnp.bfloat16)
    y = jnp.dot(h, wf_ref[1024:1280, :10],
                preferred_element_type=jnp.float32)
    y = y + f2b
    m = jnp.max(y, axis=-1, keepdims=True)
    s = y - m
    lse = jnp.log(jnp.sum(jnp.exp(s), axis=-1, keepdims=True))
    o_ref[...] = (s - lse).astype(o_ref.dtype)


def _spread_conv1(w1):
    w15 = w1.reshape(5, 5, 20)
    s1 = jnp.transpose(w15, (1, 0, 2))
    blocks = []
    for a in (0, 1):
        for b in (0, 1):
            v = _wspread(s1, 12, 28, b)
            v = jnp.transpose(v, (2, 0, 1, 3))
            t = _wspread(v, 2, 8, a)
            t = jnp.transpose(t, (1, 3, 0, 2, 4))
            blocks.append(jnp.pad(t.reshape(224, 480), ((0, 0), (0, 32))))
    return jnp.concatenate(blocks, axis=1).astype(jnp.bfloat16)


def _spread_conv2(w2):
    w25 = w2.reshape(5, 5, 20, 50)
    s1 = jnp.transpose(w25, (1, 0, 2, 3))
    blocks = []
    for a in (0, 1):
        for b in (0, 1):
            v = _wspread(s1, 4, 12, b)
            v = jnp.pad(v, ((0, 0), (0, 0), (a, 1 - a), (0, 0), (0, 0)))
            t = jnp.transpose(v, (2, 1, 3, 0, 4))
            t = t.reshape(3, 480, 200)
            t = jnp.pad(t, ((0, 0), (0, 32), (0, 56)))
            blocks.append(t.reshape(1536, 256))
    return jnp.concatenate(blocks, axis=1).astype(jnp.bfloat16)


def kernel(x, w1, b1, w2, b2, fc1_w, fc1_b, fc2_w, fc2_b):
    B = x.shape[0]
    xp = x[:, 0].reshape(B, 784)

    W1 = _spread_conv1(w1)
    W2 = _spread_conv2(w2)
    f1w = jnp.pad(fc1_w.reshape(4, 200, 256),
                  ((0, 0), (0, 56), (0, 0))).reshape(1024, 256)
    wf = jnp.concatenate(
        [f1w, jnp.pad(fc2_w, ((0, 0), (0, 246)))],
        axis=0).astype(jnp.bfloat16)
    biases = jnp.concatenate(
        [jnp.pad(jnp.tile(b1, (1, 24)), ((0, 0), (0, 32))),
         jnp.pad(jnp.tile(b2, (1, 4)), ((0, 0), (0, 56))),
         fc1_b, fc2_b], axis=1)

    TB = next((t for t in (256, 128, 64, 32, 16, 8) if B % t == 0), B)
    row_spec = lambda shape: pl.BlockSpec(shape, lambda i: (i, 0))
    full_spec = lambda shape: pl.BlockSpec(shape, lambda i: (0, 0))

    return pl.pallas_call(
        _net_kernel,
        out_shape=jax.ShapeDtypeStruct((B, 10), jnp.float32),
        grid=(B // TB,),
        in_specs=[
            row_spec((TB, 784)),
            full_spec(W1.shape),
            full_spec(W2.shape),
            full_spec(wf.shape),
            full_spec(biases.shape),
        ],
        out_specs=row_spec((TB, 10)),
        compiler_params=pltpu.CompilerParams(
            dimension_semantics=("parallel",),
            vmem_limit_bytes=100 * 1024 * 1024,
        ),
    )(xp, W1, W2, wf, biases)


# speedup vs baseline: 1.2464x; 1.2199x over previous
import jax
import jax.numpy as jnp
from jax.experimental import pallas as pl
from jax.experimental.pallas import tpu as pltpu


def _wspread(src, rows, out_w, shift):
    taps = src.shape[0]
    width = out_w + 2
    trail = src.shape[1:]
    bc = jnp.broadcast_to(src[None], (rows,) + src.shape)
    bc = jnp.pad(bc.reshape(rows, taps, -1),
                 ((0, 0), (0, width - taps), (0, 0)))
    flat = bc.reshape(rows * width, -1)
    flat = jnp.pad(flat, ((shift, 0), (0, 0)))[:rows * out_w]
    return flat.reshape((rows, out_w) + trail)


def _net_kernel(x_ref, w1_ref, w2_ref, wf_ref, bias_ref, o_ref):
    tb = x_ref.shape[0]
    xh = x_ref[...]
    b1row = bias_ref[:, :512]
    b2row = bias_ref[:, 512:768]
    f1b = bias_ref[:, 768:1024]
    f2b = bias_ref[:, 1024:1034]

    x1 = jnp.concatenate([xh[:, 128 * g:128 * g + 256] for g in range(6)],
                         axis=0)
    z1 = jnp.dot(x1, w1_ref[...], preferred_element_type=jnp.float32)
    a1_bands = []
    for g in range(6):
        z = z1[tb * g:tb * (g + 1)]
        m = jnp.maximum(jnp.maximum(z[:, :512], z[:, 512:1024]),
                        jnp.maximum(z[:, 1024:1536], z[:, 1536:]))
        a1_bands.append(
            jnp.maximum(m + b1row, 0.0).astype(jnp.bfloat16))
    a1 = jnp.concatenate(a1_bands, axis=1)

    l2 = jnp.concatenate([a1[:, 512 * y:512 * y + 1536] for y in range(4)],
                         axis=0)
    z2 = jnp.dot(l2, w2_ref[...], preferred_element_type=jnp.float32)
    a2_bands = []
    for y in range(4):
        z = z2[tb * y:tb * (y + 1)]
        m = jnp.maximum(jnp.maximum(z[:, :256], z[:, 256:512]),
                        jnp.maximum(z[:, 512:768], z[:, 768:]))
        a2_bands.append(
            jnp.maximum(m + b2row, 0.0).astype(jnp.bfloat16))
    a2 = jnp.concatenate(a2_bands, axis=1)

    h = jnp.dot(a2, wf_ref[:1024], preferred_element_type=jnp.float32)
    h = jnp.maximum(h + f1b, 0.0).astype(jnp.bfloat16)
    y = jnp.dot(h, wf_ref[1024:1280, :10],
                preferred_element_type=jnp.float32)
    y = y + f2b
    m = jnp.max(y, axis=-1, keepdims=True)
    s = y - m
    lse = jnp.log(jnp.sum(jnp.exp(s), axis=-1, keepdims=True))
    o_ref[...] = (s - lse).astype(o_ref.dtype)


def _spread_conv1(w1):
    w15 = w1.reshape(5, 5, 20)
    s1 = jnp.transpose(w15, (1, 0, 2))
    blocks = []
    for a in (0, 1):
        for b in (0, 1):
            v = _wspread(s1, 12, 28, b)
            v = jnp.transpose(v, (2, 0, 1, 3))
            t = _wspread(v, 2, 8, a)
            t = jnp.transpose(t, (1, 3, 0, 2, 4))
            t = jnp.pad(t.reshape(8, 28, 480), ((0, 0), (0, 4), (0, 32)))
            blocks.append(t.reshape(256, 512))
    return jnp.concatenate(blocks, axis=1).astype(jnp.bfloat16)


def _spread_conv2(w2):
    w25 = w2.reshape(5, 5, 20, 50)
    s1 = jnp.transpose(w25, (1, 0, 2, 3))
    blocks = []
    for a in (0, 1):
        for b in (0, 1):
            v = _wspread(s1, 4, 12, b)
            v = jnp.pad(v, ((0, 0), (0, 0), (a, 1 - a), (0, 0), (0, 0)))
            t = jnp.transpose(v, (2, 1, 3, 0, 4))
            t = t.reshape(3, 480, 200)
            t = jnp.pad(t, ((0, 0), (0, 32), (0, 56)))
            blocks.append(t.reshape(1536, 256))
    return jnp.concatenate(blocks, axis=1).astype(jnp.bfloat16)


def kernel(x, w1, b1, w2, b2, fc1_w, fc1_b, fc2_w, fc2_b):
    B = x.shape[0]
    xp = jnp.pad(x.reshape(B, 28, 28), ((0, 0), (0, 0), (0, 4)))
    xp = xp.reshape(B, 896).astype(jnp.bfloat16)

    W1 = _spread_conv1(w1)
    W2 = _spread_conv2(w2)
    f1w = jnp.pad(fc1_w.reshape(4, 200, 256),
                  ((0, 0), (0, 56), (0, 0))).reshape(1024, 256)
    wf = jnp.concatenate(
        [f1w, jnp.pad(fc2_w, ((0, 0), (0, 246)))],
        axis=0).astype(jnp.bfloat16)
    biases = jnp.concatenate(
        [jnp.pad(jnp.tile(b1, (1, 24)), ((0, 0), (0, 32))),
         jnp.pad(jnp.tile(b2, (1, 4)), ((0, 0), (0, 56))),
         fc1_b, fc2_b], axis=1)

    TB = next((t for t in (256, 128, 64, 32, 16, 8) if B % t == 0), B)
    row_spec = lambda shape: pl.BlockSpec(shape, lambda i: (i, 0))
    full_spec = lambda shape: pl.BlockSpec(shape, lambda i: (0, 0))

    return pl.pallas_call(
        _net_kernel,
        out_shape=jax.ShapeDtypeStruct((B, 10), jnp.float32),
        grid=(B // TB,),
        in_specs=[
            row_spec((TB, 896)),
            full_spec(W1.shape),
            full_spec(W2.shape),
            full_spec(wf.shape),
            full_spec(biases.shape),
        ],
        out_specs=row_spec((TB, 10)),
        compiler_params=pltpu.CompilerParams(
            dimension_semantics=("parallel",),
            vmem_limit_bytes=100 * 1024 * 1024,
        ),
    )(xp, W1, W2, wf, biases)


# speedup vs baseline: 1.2707x; 1.0195x over previous
import jax
import jax.numpy as jnp
from jax.experimental import pallas as pl
from jax.experimental.pallas import tpu as pltpu


def _wspread(src, rows, out_w, shift):
    taps = src.shape[0]
    width = out_w + 2
    trail = src.shape[1:]
    bc = jnp.broadcast_to(src[None], (rows,) + src.shape)
    bc = jnp.pad(bc.reshape(rows, taps, -1),
                 ((0, 0), (0, width - taps), (0, 0)))
    flat = bc.reshape(rows * width, -1)
    flat = jnp.pad(flat, ((shift, 0), (0, 0)))[:rows * out_w]
    return flat.reshape((rows, out_w) + trail)


def _net_kernel(x_ref, w1_ref, w2_ref, wf_ref, bias_ref, o_ref):
    tb = x_ref.shape[0]
    xh = x_ref[...]
    b1row = bias_ref[:, :512]
    b2row = bias_ref[:, 512:768]
    f1b = bias_ref[:, 768:1024]
    f2b = bias_ref[:, 1024:1034]

    x1 = jnp.concatenate([xh[:, 128 * g:128 * g + 256] for g in range(6)],
                         axis=0)
    z1 = jnp.dot(x1, w1_ref[...], preferred_element_type=jnp.float32)
    a1_bands = []
    for g in range(6):
        z = z1[tb * g:tb * (g + 1)]
        m = jnp.maximum(jnp.maximum(z[:, :512], z[:, 512:1024]),
                        jnp.maximum(z[:, 1024:1536], z[:, 1536:]))
        a1_bands.append(
            jnp.maximum(m + b1row, 0.0).astype(jnp.bfloat16))
    a1 = jnp.concatenate(a1_bands, axis=1)

    l2 = jnp.concatenate([a1[:, 512 * y:512 * y + 1536] for y in range(4)],
                         axis=0)
    z2 = jnp.dot(l2, w2_ref[...], preferred_element_type=jnp.float32)
    a2_bands = []
    for y in range(4):
        z = z2[tb * y:tb * (y + 1)]
        m = jnp.maximum(jnp.maximum(z[:, :256], z[:, 256:512]),
                        jnp.maximum(z[:, 512:768], z[:, 768:]))
        a2_bands.append(
            jnp.maximum(m + b2row, 0.0).astype(jnp.bfloat16))
    a2 = jnp.concatenate(a2_bands, axis=1)

    h = jnp.dot(a2, wf_ref[:1024], preferred_element_type=jnp.float32)
    h = jnp.maximum(h + f1b, 0.0).astype(jnp.bfloat16)
    y = jnp.dot(h, wf_ref[1024:1280, :10],
                preferred_element_type=jnp.float32)
    y = y + f2b
    m = jnp.max(y, axis=-1, keepdims=True)
    s = y - m
    lse = jnp.log(jnp.sum(jnp.exp(s), axis=-1, keepdims=True))
    o_ref[...] = (s - lse).astype(o_ref.dtype)


def _spread_conv1(w1):
    w15 = w1.reshape(5, 5, 20)
    s1 = jnp.transpose(w15, (1, 0, 2))
    blocks = []
    for a in (0, 1):
        for b in (0, 1):
            v = _wspread(s1, 12, 28, b)
            v = jnp.transpose(v, (2, 0, 1, 3))
            t = _wspread(v, 2, 8, a)
            t = jnp.transpose(t, (1, 3, 0, 2, 4))
            t = jnp.pad(t.reshape(8, 28, 480), ((0, 0), (0, 4), (0, 32)))
            blocks.append(t.reshape(256, 512))
    return jnp.concatenate(blocks, axis=1).astype(jnp.bfloat16)


def _spread_conv2(w2):
    w25 = w2.reshape(5, 5, 20, 50)
    s1 = jnp.transpose(w25, (1, 0, 2, 3))
    blocks = []
    for a in (0, 1):
        for b in (0, 1):
            v = _wspread(s1, 4, 12, b)
            v = jnp.pad(v, ((0, 0), (0, 0), (a, 1 - a), (0, 0), (0, 0)))
            t = jnp.transpose(v, (2, 1, 3, 0, 4))
            t = t.reshape(3, 480, 200)
            t = jnp.pad(t, ((0, 0), (0, 32), (0, 56)))
            blocks.append(t.reshape(1536, 256))
    return jnp.concatenate(blocks, axis=1).astype(jnp.bfloat16)


def kernel(x, w1, b1, w2, b2, fc1_w, fc1_b, fc2_w, fc2_b):
    B = x.shape[0]
    xp = jnp.pad(x.reshape(B, 28, 28), ((0, 0), (0, 0), (0, 4)))
    xp = xp.reshape(B, 896).astype(jnp.bfloat16)

    W1 = _spread_conv1(w1)
    W2 = _spread_conv2(w2)
    f1w = jnp.pad(fc1_w.reshape(4, 200, 256),
                  ((0, 0), (0, 56), (0, 0))).reshape(1024, 256)
    wf = jnp.concatenate(
        [f1w, jnp.pad(fc2_w, ((0, 0), (0, 246)))],
        axis=0).astype(jnp.bfloat16)
    biases = jnp.concatenate(
        [jnp.pad(jnp.tile(b1, (1, 24)), ((0, 0), (0, 32))),
         jnp.pad(jnp.tile(b2, (1, 4)), ((0, 0), (0, 56))),
         fc1_b, fc2_b], axis=1)

    TB = next((t for t in (512, 256, 128, 64, 32, 16, 8) if B % t == 0), B)
    row_spec = lambda shape: pl.BlockSpec(shape, lambda i: (i, 0))
    full_spec = lambda shape: pl.BlockSpec(shape, lambda i: (0, 0))

    return pl.pallas_call(
        _net_kernel,
        out_shape=jax.ShapeDtypeStruct((B, 10), jnp.float32),
        grid=(B // TB,),
        in_specs=[
            row_spec((TB, 896)),
            full_spec(W1.shape),
            full_spec(W2.shape),
            full_spec(wf.shape),
            full_spec(biases.shape),
        ],
        out_specs=row_spec((TB, 10)),
        compiler_params=pltpu.CompilerParams(
            dimension_semantics=("parallel",),
            vmem_limit_bytes=100 * 1024 * 1024,
        ),
    )(xp, W1, W2, wf, biases)
